# baseline (speedup 1.0000x reference)
import functools

import jax
import jax.numpy as jnp
from jax import lax
from jax.experimental import pallas as pl
from jax.experimental.pallas import tpu as pltpu
from jax.experimental.pallas import tpu_sc as plsc

_N = 10000
_E = 320000
_D = 128
_HID = 16
_HEADS = 8
_C = 32

_NCORES = 2
_NSUB = 16
_PACK = 16384


def _offs16(n):
    o = list(range(0, n - 15, 16))
    if n % 16:
        o.append(n - 16)
    return o


def _tc1_body(x_ref, w_ref, a_ref, p_ref, hb_ref, asd_ref):
    h = jnp.dot(x_ref[...], w_ref[...], preferred_element_type=jnp.float32)
    asd_ref[...] = jnp.dot(h, a_ref[...], preferred_element_type=jnp.float32)
    hb_ref[...] = jnp.dot(h, p_ref[...],
                          preferred_element_type=jnp.float32
                          ).astype(jnp.bfloat16)


def _tc1(x, W1, A1, P1):
    blk = 2000
    grid = _N // blk
    return pl.pallas_call(
        _tc1_body,
        grid=(grid,),
        in_specs=[
            pl.BlockSpec((blk, _D), lambda i: (i, 0)),
            pl.BlockSpec((_D, _D), lambda i: (0, 0)),
            pl.BlockSpec((_D, 16), lambda i: (0, 0)),
            pl.BlockSpec((_D, _D), lambda i: (0, 0)),
        ],
        out_specs=[
            pl.BlockSpec((blk, _D), lambda i: (i, 0)),
            pl.BlockSpec((blk, 16), lambda i: (i, 0)),
        ],
        out_shape=[
            jax.ShapeDtypeStruct((_N, _D), jnp.bfloat16),
            jax.ShapeDtypeStruct((_N, 16), jnp.float32),
        ],
    )(x, W1, A1, P1)


def _tc2_body(a0_ref, a1_ref, b1_ref, w2_ref, a2_ref, r1_ref, p2_ref,
              hb2_ref, asd2_ref):
    tot = a0_ref[0] + a1_ref[0]
    s = tot[:, _D:_D + _HEADS]
    recip = 1.0 / (s + 1e-16)
    rexp = jnp.dot(recip, r1_ref[...], preferred_element_type=jnp.float32)
    z = tot[:, :_D] * rexp + b1_ref[...]
    h2in = jnp.where(z > 0, z, jnp.exp(z) - 1.0)
    h2 = jnp.dot(h2in, w2_ref[...], preferred_element_type=jnp.float32)
    asd2_ref[...] = jnp.dot(h2, a2_ref[...], preferred_element_type=jnp.float32)
    hb2_ref[...] = jnp.dot(h2, p2_ref[...],
                           preferred_element_type=jnp.float32
                           ).astype(jnp.bfloat16)


def _tc2(acc, b1, W2, A2, R1, P2):
    blk = 2000
    grid = _N // blk
    row = _D + 8
    return pl.pallas_call(
        _tc2_body,
        grid=(grid,),
        in_specs=[
            pl.BlockSpec((1, blk, row), lambda i: (0, i, 0)),
            pl.BlockSpec((1, blk, row), lambda i: (1, i, 0)),
            pl.BlockSpec((1, _D), lambda i: (0, 0)),
            pl.BlockSpec((_D, _C), lambda i: (0, 0)),
            pl.BlockSpec((_C, 16), lambda i: (0, 0)),
            pl.BlockSpec((_HEADS, _D), lambda i: (0, 0)),
            pl.BlockSpec((_C, _C), lambda i: (0, 0)),
        ],
        out_specs=[
            pl.BlockSpec((blk, _C), lambda i: (i, 0)),
            pl.BlockSpec((blk, 16), lambda i: (i, 0)),
        ],
        out_shape=[
            jax.ShapeDtypeStruct((_N, _C), jnp.bfloat16),
            jax.ShapeDtypeStruct((_N, 16), jnp.float32),
        ],
    )(acc, acc, b1, W2, A2, R1, P2)


def _tc3_body(a0_ref, a1_ref, b2_ref, r2_ref, out_ref):
    tot = a0_ref[0] + a1_ref[0]
    s = tot[:, _C:_C + 1]
    recip = 1.0 / (s + 1e-16)
    rexp = jnp.dot(recip, r2_ref[...], preferred_element_type=jnp.float32)
    z = tot[:, :_C] * rexp + b2_ref[...]
    z = jnp.where(z > 0, z, jnp.exp(z) - 1.0)
    m = jnp.max(z, axis=1, keepdims=True)
    lse = jnp.log(jnp.sum(jnp.exp(z - m), axis=1, keepdims=True)) + m
    out_ref[...] = z - lse


def _tc3(acc, b2, R2):
    blk = 2000
    grid = _N // blk
    row = _C + 8
    return pl.pallas_call(
        _tc3_body,
        grid=(grid,),
        in_specs=[
            pl.BlockSpec((1, blk, row), lambda i: (0, i, 0)),
            pl.BlockSpec((1, blk, row), lambda i: (1, i, 0)),
            pl.BlockSpec((1, _C), lambda i: (0, 0)),
            pl.BlockSpec((1, _C), lambda i: (0, 0)),
        ],
        out_specs=pl.BlockSpec((blk, _C), lambda i: (i, 0)),
        out_shape=jax.ShapeDtypeStruct((_N, _C), jnp.float32),
    )(acc, acc, b2, R2)


def _make_sc_layer(F, heads, CH):
    ROW = F + 8
    NW = _NCORES * _NSUB
    EPW = _E // NW
    NCH = EPW // CH
    NR = NCH // 4
    ZR = 40
    NB = _N // ZR
    NBT = -(-NB // _NSUB)
    fseg = F // 16
    roffs = _offs16(ROW)
    coffs = _offs16(CH)
    mesh = plsc.VectorSubcoreMesh(core_axis_name="c", subcore_axis_name="s")

    @functools.partial(
        pl.kernel, mesh=mesh,
        compiler_params=pltpu.CompilerParams(use_tc_tiling_on_sc=False,
                                             needs_layout_passes=False),
        out_type=jax.ShapeDtypeStruct((_NCORES, _N, ROW), jnp.float32),
        scratch_types=[
            pltpu.VMEM((NCH, CH), jnp.int32),
            [pltpu.VMEM((1, CH), jnp.int32) for _ in range(4)],
            [pltpu.VMEM((1, CH), jnp.int32) for _ in range(4)],
            [pltpu.VMEM((CH, 16), jnp.float32) for _ in range(2)],
            [pltpu.VMEM((CH, 16), jnp.float32) for _ in range(2)],
            [pltpu.VMEM((CH, F), jnp.bfloat16) for _ in range(2)],
            [pltpu.VMEM((CH, ROW), jnp.float32) for _ in range(2)],
            pltpu.VMEM_SHARED((_N, ROW), jnp.float32),
            [pltpu.SemaphoreType.DMA for _ in range(2)],
            [pltpu.SemaphoreType.DMA for _ in range(2)],
        ],
    )
    def sck(h_hbm, asd_hbm, pk_hbm, out_hbm,
            pk_all, sidx, didx, asv, adv, rv, mv, acc_sh, gsem, ssem):
        c = lax.axis_index("c")
        s = lax.axis_index("s")
        w = c * _NSUB + s
        lane = lax.iota(jnp.int32, 16)
        zero16 = jnp.zeros((16,), jnp.float32)

        pltpu.sync_copy(pk_hbm.at[pl.ds(w * NCH, NCH)], pk_all)

        def zrow(r, carry):
            for k in roffs:
                mv[0][r, pl.ds(k, 16)] = zero16
            return carry
        lax.fori_loop(0, ZR, zrow, 0)

        def zblk(ii, carry):
            bi = s + ii * _NSUB

            @pl.when(bi < NB)
            def _():
                off = pl.multiple_of(bi * ZR, 8)
                pltpu.sync_copy(mv[0].at[pl.ds(0, ZR)],
                                acc_sh.at[pl.ds(off, ZR)])
            return carry
        lax.fori_loop(0, NBT, zblk, 0)
        plsc.subcore_barrier()

        def unpack(i, islot):
            for off in coffs:
                pk = pk_all[i, pl.ds(off, 16)]
                sidx[islot][0, pl.ds(off, 16)] = jnp.bitwise_and(pk, _PACK - 1)
                didx[islot][0, pl.ds(off, 16)] = lax.shift_right_logical(
                    pk, 14)

        def gathers(dslot, islot):
            idx = sidx[islot].at[0]
            return (
                pltpu.make_async_copy(asd_hbm.at[idx], asv[dslot],
                                      gsem[dslot]),
                pltpu.make_async_copy(asd_hbm.at[didx[islot].at[0]],
                                      adv[dslot], gsem[dslot]),
                pltpu.make_async_copy(h_hbm.at[idx], rv[dslot], gsem[dslot]),
            )

        def scatter(dslot, islot):
            return pltpu.make_async_copy(mv[dslot],
                                         acc_sh.at[didx[islot].at[0]],
                                         ssem[dslot])

        def compute(dslot):
            @plsc.parallel_loop(0, CH, 1, unroll=8)
            def edge(j):
                a_s = asv[dslot][j]
                a_d = adv[dslot][j]
                if heads == 8:
                    shift = jnp.where(lane < 8, lane + 8, lane)
                    ad = a_d.at[shift].get(mode="promise_in_bounds")
                    e = a_s + ad
                else:
                    asb = a_s.at[jnp.zeros((16,), jnp.int32)].get(
                        mode="promise_in_bounds")
                    adb = a_d.at[jnp.ones((16,), jnp.int32)].get(
                        mode="promise_in_bounds")
                    e = asb + adb
                e = jnp.where(e >= 0, e, 0.2 * e)
                p = jnp.exp(e)
                last = None
                for k2 in range(F // 32):
                    v = rv[dslot][j, pl.ds(k2 * 32, 32)]
                    a, b = plsc.unpack(v, format=plsc.PackFormat.INTERLEAVED)
                    if heads == 8:
                        pa = p.at[jnp.full((16,), 2 * k2, jnp.int32)].get(
                            mode="promise_in_bounds")
                        pb = p.at[jnp.full((16,), 2 * k2 + 1, jnp.int32)].get(
                            mode="promise_in_bounds")
                    else:
                        pa = pb = p
                    mv[dslot][j, pl.ds(k2 * 32, 16)] = a * pa
                    last = b * pb
                    mv[dslot][j, pl.ds(k2 * 32 + 16, 16)] = last
                lhalf = last.at[jnp.bitwise_and(lane + 8, 15)].get(
                    mode="promise_in_bounds")
                if heads == 8:
                    phalf = p.at[jnp.bitwise_and(lane + 8, 15)].get(
                        mode="promise_in_bounds")
                else:
                    phalf = jnp.where(lane == 8, p, 0.0)
                mv[dslot][j, pl.ds(F - 8, 16)] = jnp.where(
                    lane < 8, lhalf, phalf)

        def half(i, dslot, islot):
            for cp in gathers(dslot, islot):
                cp.wait()

            @pl.when(i >= 2)
            def _():
                scatter(dslot, (islot + 2) % 4).wait()
            compute(dslot)
            scatter(dslot, islot).start(add=True)

            @pl.when(i + 2 < NCH)
            def _():
                unpack(i + 2, (islot + 2) % 4)
                for cp in gathers(dslot, (islot + 2) % 4):
                    cp.start()

        unpack(0, 0)
        for cp in gathers(0, 0):
            cp.start()
        unpack(1, 1)
        for cp in gathers(1, 1):
            cp.start()

        def ring(r, carry):
            i = 4 * r
            half(i, 0, 0)
            half(i + 1, 1, 1)
            half(i + 2, 0, 2)
            half(i + 3, 1, 3)
            return carry
        lax.fori_loop(0, NR, ring, 0)
        scatter(0, 2).wait()
        scatter(1, 3).wait()
        plsc.subcore_barrier()

        def cblk(ii, carry):
            bi = s + ii * _NSUB

            @pl.when(bi < NB)
            def _():
                off = pl.multiple_of(bi * ZR, 8)
                pltpu.sync_copy(acc_sh.at[pl.ds(off, ZR)],
                                out_hbm.at[c, pl.ds(off, ZR)])
            return carry
        lax.fori_loop(0, NBT, cblk, 0)

    return sck


_CH1 = 50
_CH2 = 125
_sc_layer1 = _make_sc_layer(_D, _HEADS, _CH1)
_sc_layer2 = _make_sc_layer(_C, 1, _CH2)


def kernel(x, edge_index, W1, a_src1, a_dst1, b1, W2, a_src2, a_dst2, b2):
    packed = edge_index[0] + edge_index[1] * _PACK
    pk1 = packed.reshape(-1, _CH1)
    pk2 = packed.reshape(-1, _CH2)

    eye8 = jnp.eye(_HEADS, dtype=jnp.float32)
    A1 = jnp.concatenate(
        [(a_src1[:, :, None] * eye8[:, None, :]).reshape(_D, _HEADS),
         (a_dst1[:, :, None] * eye8[:, None, :]).reshape(_D, _HEADS)], axis=1)
    A2 = jnp.concatenate(
        [a_src2.T, a_dst2.T, jnp.zeros((_C, 14), jnp.float32)], axis=1)
    R1 = (jnp.arange(_D, dtype=jnp.int32)[None, :] // _HID
          == jnp.arange(_HEADS, dtype=jnp.int32)[:, None]).astype(jnp.float32)
    R2 = jnp.ones((1, _C), jnp.float32)

    def interleave_perm(F):
        o = jnp.arange(F, dtype=jnp.int32)
        grp, r = o // 32, o % 32
        new = jnp.where(r < 16, grp * 32 + 2 * r, grp * 32 + 2 * (r - 16) + 1)
        return (new[:, None] == jnp.arange(F, dtype=jnp.int32)[None, :]
                ).astype(jnp.float32)
    P1 = interleave_perm(_D)
    P2 = interleave_perm(_C)

    hb1, asd1 = _tc1(x, W1, A1, P1)
    acc1 = _sc_layer1(hb1, asd1, pk1)
    hb2, asd2 = _tc2(acc1, b1.reshape(1, _D), W2, A2, R1, P2)
    acc2 = _sc_layer2(hb2, asd2, pk2)
    out = _tc3(acc2, b2.reshape(1, _C), R2)
    return out


# speedup vs baseline: 1.0059x; 1.0059x over previous
import functools

import jax
import jax.numpy as jnp
from jax import lax
from jax.experimental import pallas as pl
from jax.experimental.pallas import tpu as pltpu
from jax.experimental.pallas import tpu_sc as plsc

_N = 10000
_E = 320000
_D = 128
_HID = 16
_HEADS = 8
_C = 32

_NCORES = 2
_NSUB = 16
_PACK = 16384


def _offs16(n):
    o = list(range(0, n - 15, 16))
    if n % 16:
        o.append(n - 16)
    return o


def _tc1_body(x_ref, w_ref, a_ref, p_ref, hb_ref, asd_ref):
    h = jnp.dot(x_ref[...], w_ref[...], preferred_element_type=jnp.float32)
    asd_ref[...] = jnp.dot(h, a_ref[...], preferred_element_type=jnp.float32)
    hb_ref[...] = jnp.dot(h, p_ref[...],
                          preferred_element_type=jnp.float32
                          ).astype(jnp.bfloat16)


def _tc1(x, W1, A1, P1):
    blk = 2000
    grid = _N // blk
    return pl.pallas_call(
        _tc1_body,
        grid=(grid,),
        in_specs=[
            pl.BlockSpec((blk, _D), lambda i: (i, 0)),
            pl.BlockSpec((_D, _D), lambda i: (0, 0)),
            pl.BlockSpec((_D, 16), lambda i: (0, 0)),
            pl.BlockSpec((_D, _D), lambda i: (0, 0)),
        ],
        out_specs=[
            pl.BlockSpec((blk, _D), lambda i: (i, 0)),
            pl.BlockSpec((blk, 16), lambda i: (i, 0)),
        ],
        out_shape=[
            jax.ShapeDtypeStruct((_N, _D), jnp.bfloat16),
            jax.ShapeDtypeStruct((_N, 16), jnp.float32),
        ],
    )(x, W1, A1, P1)


def _tc2_body(a0_ref, a1_ref, b1_ref, w2_ref, a2_ref, r1_ref, p2_ref,
              hb2_ref, asd2_ref):
    tot = a0_ref[0] + a1_ref[0]
    s = tot[:, _D:_D + _HEADS]
    recip = 1.0 / (s + 1e-16)
    rexp = jnp.dot(recip, r1_ref[...], preferred_element_type=jnp.float32)
    z = tot[:, :_D] * rexp + b1_ref[...]
    h2in = jnp.where(z > 0, z, jnp.exp(z) - 1.0)
    h2 = jnp.dot(h2in, w2_ref[...], preferred_element_type=jnp.float32)
    asd2_ref[...] = jnp.dot(h2, a2_ref[...], preferred_element_type=jnp.float32)
    hb2_ref[...] = jnp.dot(h2, p2_ref[...],
                           preferred_element_type=jnp.float32
                           ).astype(jnp.bfloat16)


def _tc2(acc, b1, W2, A2, R1, P2):
    blk = 2000
    grid = _N // blk
    row = _D + 8
    return pl.pallas_call(
        _tc2_body,
        grid=(grid,),
        in_specs=[
            pl.BlockSpec((1, blk, row), lambda i: (0, i, 0)),
            pl.BlockSpec((1, blk, row), lambda i: (1, i, 0)),
            pl.BlockSpec((1, _D), lambda i: (0, 0)),
            pl.BlockSpec((_D, _C), lambda i: (0, 0)),
            pl.BlockSpec((_C, 16), lambda i: (0, 0)),
            pl.BlockSpec((_HEADS, _D), lambda i: (0, 0)),
            pl.BlockSpec((_C, _C), lambda i: (0, 0)),
        ],
        out_specs=[
            pl.BlockSpec((blk, _C), lambda i: (i, 0)),
            pl.BlockSpec((blk, 16), lambda i: (i, 0)),
        ],
        out_shape=[
            jax.ShapeDtypeStruct((_N, _C), jnp.bfloat16),
            jax.ShapeDtypeStruct((_N, 16), jnp.float32),
        ],
    )(acc, acc, b1, W2, A2, R1, P2)


def _tc3_body(a0_ref, a1_ref, b2_ref, r2_ref, out_ref):
    tot = a0_ref[0] + a1_ref[0]
    s = tot[:, _C:_C + 1]
    recip = 1.0 / (s + 1e-16)
    rexp = jnp.dot(recip, r2_ref[...], preferred_element_type=jnp.float32)
    z = tot[:, :_C] * rexp + b2_ref[...]
    z = jnp.where(z > 0, z, jnp.exp(z) - 1.0)
    m = jnp.max(z, axis=1, keepdims=True)
    lse = jnp.log(jnp.sum(jnp.exp(z - m), axis=1, keepdims=True)) + m
    out_ref[...] = z - lse


def _tc3(acc, b2, R2):
    blk = 2000
    grid = _N // blk
    row = _C + 8
    return pl.pallas_call(
        _tc3_body,
        grid=(grid,),
        in_specs=[
            pl.BlockSpec((1, blk, row), lambda i: (0, i, 0)),
            pl.BlockSpec((1, blk, row), lambda i: (1, i, 0)),
            pl.BlockSpec((1, _C), lambda i: (0, 0)),
            pl.BlockSpec((1, _C), lambda i: (0, 0)),
        ],
        out_specs=pl.BlockSpec((blk, _C), lambda i: (i, 0)),
        out_shape=jax.ShapeDtypeStruct((_N, _C), jnp.float32),
    )(acc, acc, b2, R2)


def _make_sc_layer(F, heads, CH):
    ROW = F + 8
    NW = _NCORES * _NSUB
    EPW = _E // NW
    NCH = EPW // CH
    NR = NCH // 4
    ZR = 40
    NB = _N // ZR
    NBT = -(-NB // _NSUB)
    fseg = F // 16
    roffs = _offs16(ROW)
    coffs = _offs16(CH)
    mesh = plsc.VectorSubcoreMesh(core_axis_name="c", subcore_axis_name="s")

    @functools.partial(
        pl.kernel, mesh=mesh,
        compiler_params=pltpu.CompilerParams(use_tc_tiling_on_sc=False,
                                             needs_layout_passes=False),
        out_type=jax.ShapeDtypeStruct((_NCORES, _N, ROW), jnp.float32),
        scratch_types=[
            pltpu.VMEM((NCH, CH), jnp.int32),
            [pltpu.VMEM((1, CH), jnp.int32) for _ in range(4)],
            [pltpu.VMEM((1, CH), jnp.int32) for _ in range(4)],
            [pltpu.VMEM((CH, 16), jnp.float32) for _ in range(2)],
            [pltpu.VMEM((CH, 16), jnp.float32) for _ in range(2)],
            [pltpu.VMEM((CH, F), jnp.bfloat16) for _ in range(2)],
            [pltpu.VMEM((CH, ROW), jnp.float32) for _ in range(2)],
            pltpu.VMEM_SHARED((_N, ROW), jnp.float32),
            [pltpu.SemaphoreType.DMA for _ in range(2)],
            [pltpu.SemaphoreType.DMA for _ in range(2)],
        ],
    )
    def sck(h_hbm, asd_hbm, pk_hbm, out_hbm,
            pk_all, sidx, didx, asv, adv, rv, mv, acc_sh, gsem, ssem):
        c = lax.axis_index("c")
        s = lax.axis_index("s")
        w = c * _NSUB + s
        lane = lax.iota(jnp.int32, 16)
        zero16 = jnp.zeros((16,), jnp.float32)

        pltpu.sync_copy(pk_hbm.at[pl.ds(w * NCH, NCH)], pk_all)

        def zrow(r, carry):
            for k in roffs:
                mv[0][r, pl.ds(k, 16)] = zero16
            return carry
        lax.fori_loop(0, ZR, zrow, 0)

        def zblk(ii, carry):
            bi = s + ii * _NSUB

            @pl.when(bi < NB)
            def _():
                off = pl.multiple_of(bi * ZR, 8)
                pltpu.sync_copy(mv[0].at[pl.ds(0, ZR)],
                                acc_sh.at[pl.ds(off, ZR)])
            return carry
        lax.fori_loop(0, NBT, zblk, 0)
        plsc.subcore_barrier()

        def unpack(i, islot):
            for off in coffs:
                pk = pk_all[i, pl.ds(off, 16)]
                sidx[islot][0, pl.ds(off, 16)] = jnp.bitwise_and(pk, _PACK - 1)
                didx[islot][0, pl.ds(off, 16)] = lax.shift_right_logical(
                    pk, 14)

        def gathers(dslot, islot):
            idx = sidx[islot].at[0]
            return (
                pltpu.make_async_copy(asd_hbm.at[idx], asv[dslot],
                                      gsem[dslot]),
                pltpu.make_async_copy(asd_hbm.at[didx[islot].at[0]],
                                      adv[dslot], gsem[dslot]),
                pltpu.make_async_copy(h_hbm.at[idx], rv[dslot], gsem[dslot]),
            )

        def scatter(dslot, islot):
            return pltpu.make_async_copy(mv[dslot],
                                         acc_sh.at[didx[islot].at[0]],
                                         ssem[dslot])

        def compute(dslot):
            @plsc.parallel_loop(0, CH, 1, unroll=4)
            def edge(j):
                a_s = asv[dslot][j]
                a_d = adv[dslot][j]
                if heads == 8:
                    shift = jnp.where(lane < 8, lane + 8, lane)
                    ad = a_d.at[shift].get(mode="promise_in_bounds")
                    e = a_s + ad
                else:
                    asb = a_s.at[jnp.zeros((16,), jnp.int32)].get(
                        mode="promise_in_bounds")
                    adb = a_d.at[jnp.ones((16,), jnp.int32)].get(
                        mode="promise_in_bounds")
                    e = asb + adb
                e = jnp.where(e >= 0, e, 0.2 * e)
                p = jnp.exp(e)
                last = None
                for k2 in range(F // 32):
                    v = rv[dslot][j, pl.ds(k2 * 32, 32)]
                    a, b = plsc.unpack(v, format=plsc.PackFormat.INTERLEAVED)
                    if heads == 8:
                        pa = p.at[jnp.full((16,), 2 * k2, jnp.int32)].get(
                            mode="promise_in_bounds")
                        pb = p.at[jnp.full((16,), 2 * k2 + 1, jnp.int32)].get(
                            mode="promise_in_bounds")
                    else:
                        pa = pb = p
                    mv[dslot][j, pl.ds(k2 * 32, 16)] = a * pa
                    last = b * pb
                    mv[dslot][j, pl.ds(k2 * 32 + 16, 16)] = last
                lhalf = last.at[jnp.bitwise_and(lane + 8, 15)].get(
                    mode="promise_in_bounds")
                if heads == 8:
                    phalf = p.at[jnp.bitwise_and(lane + 8, 15)].get(
                        mode="promise_in_bounds")
                else:
                    phalf = jnp.where(lane == 8, p, 0.0)
                mv[dslot][j, pl.ds(F - 8, 16)] = jnp.where(
                    lane < 8, lhalf, phalf)

        def half(i, dslot, islot):
            for cp in gathers(dslot, islot):
                cp.wait()

            @pl.when(i >= 2)
            def _():
                scatter(dslot, (islot + 2) % 4).wait()
            compute(dslot)
            scatter(dslot, islot).start(add=True)

            @pl.when(i + 2 < NCH)
            def _():
                unpack(i + 2, (islot + 2) % 4)
                for cp in gathers(dslot, (islot + 2) % 4):
                    cp.start()

        unpack(0, 0)
        for cp in gathers(0, 0):
            cp.start()
        unpack(1, 1)
        for cp in gathers(1, 1):
            cp.start()

        def ring(r, carry):
            i = 4 * r
            half(i, 0, 0)
            half(i + 1, 1, 1)
            half(i + 2, 0, 2)
            half(i + 3, 1, 3)
            return carry
        lax.fori_loop(0, NR, ring, 0)
        scatter(0, 2).wait()
        scatter(1, 3).wait()
        plsc.subcore_barrier()

        def cblk(ii, carry):
            bi = s + ii * _NSUB

            @pl.when(bi < NB)
            def _():
                off = pl.multiple_of(bi * ZR, 8)
                pltpu.sync_copy(acc_sh.at[pl.ds(off, ZR)],
                                out_hbm.at[c, pl.ds(off, ZR)])
            return carry
        lax.fori_loop(0, NBT, cblk, 0)

    return sck


_CH1 = 50
_CH2 = 125
_sc_layer1 = _make_sc_layer(_D, _HEADS, _CH1)
_sc_layer2 = _make_sc_layer(_C, 1, _CH2)


def kernel(x, edge_index, W1, a_src1, a_dst1, b1, W2, a_src2, a_dst2, b2):
    packed = edge_index[0] + edge_index[1] * _PACK
    pk1 = packed.reshape(-1, _CH1)
    pk2 = packed.reshape(-1, _CH2)

    eye8 = jnp.eye(_HEADS, dtype=jnp.float32)
    A1 = jnp.concatenate(
        [(a_src1[:, :, None] * eye8[:, None, :]).reshape(_D, _HEADS),
         (a_dst1[:, :, None] * eye8[:, None, :]).reshape(_D, _HEADS)], axis=1)
    A2 = jnp.concatenate(
        [a_src2.T, a_dst2.T, jnp.zeros((_C, 14), jnp.float32)], axis=1)
    R1 = (jnp.arange(_D, dtype=jnp.int32)[None, :] // _HID
          == jnp.arange(_HEADS, dtype=jnp.int32)[:, None]).astype(jnp.float32)
    R2 = jnp.ones((1, _C), jnp.float32)

    def interleave_perm(F):
        o = jnp.arange(F, dtype=jnp.int32)
        grp, r = o // 32, o % 32
        new = jnp.where(r < 16, grp * 32 + 2 * r, grp * 32 + 2 * (r - 16) + 1)
        return (new[:, None] == jnp.arange(F, dtype=jnp.int32)[None, :]
                ).astype(jnp.float32)
    P1 = interleave_perm(_D)
    P2 = interleave_perm(_C)

    hb1, asd1 = _tc1(x, W1, A1, P1)
    acc1 = _sc_layer1(hb1, asd1, pk1)
    hb2, asd2 = _tc2(acc1, b1.reshape(1, _D), W2, A2, R1, P2)
    acc2 = _sc_layer2(hb2, asd2, pk2)
    out = _tc3(acc2, b2.reshape(1, _C), R2)
    return out
